# Initial kernel scaffold; baseline (speedup 1.0000x reference)
import jax
import jax.numpy as jnp
from jax.experimental import pallas as pl


def kernel(pointclouds, valid_points):
    raise NotImplementedError("write your pallas kernel here")


# speedup vs baseline: 1.4511x; 1.4511x over previous
import functools

import jax
import jax.numpy as jnp
from jax import lax
from jax.experimental import pallas as pl
from jax.experimental.pallas import tpu as pltpu
from jax.experimental.pallas import tpu_sc as plsc

B = 16
P = 131072
RES = 50
NVOX = RES * RES * RES
GRID_PAD = 125008
OUT_ROW = NVOX * 4

NC, NS, L = 2, 16, 16

PCHUNK = 1024


BB = 8


def _enc_body(pts_ref, valid_ref, enc_ref):
    pts = pts_ref[...]

    def coord(c):
        t = (pts[:, :, c] + 2.0) * 0.25 * 49.0
        ti = jnp.floor(t).astype(jnp.int32)
        return jnp.clip(ti, 0, 49)

    i, j, k = coord(0), coord(1), coord(2)
    lin = (i * 50 + j) * 50 + k
    enc_ref[...] = jnp.where(valid_ref[...], lin, -1)


def _encode(pointclouds, valid):
    return pl.pallas_call(
        _enc_body,
        grid=(B // BB, P // PCHUNK),
        in_specs=[
            pl.BlockSpec((BB, PCHUNK, 6), lambda b, c: (b, c, 0)),
            pl.BlockSpec((BB, PCHUNK), lambda b, c: (b, c)),
        ],
        out_specs=pl.BlockSpec((BB, PCHUNK), lambda b, c: (b, c)),
        out_shape=jax.ShapeDtypeStruct((B, P), jnp.int32),
    )(pointclouds, valid)


ECHUNK = 2048
NECH = P // ECHUNK
XCHUNK = 512
XOUT = XCHUNK * 4
NXFULL = NVOX // XCHUNK
TAIL_GRID = NVOX - NXFULL * XCHUNK
TAIL_OUT = TAIL_GRID * 4

_mesh = plsc.VectorSubcoreMesh(
    core_axis_name="c", subcore_axis_name="s", num_cores=NC, num_subcores=NS)


ONE_F32_BITS = 0x3F800000


@functools.partial(
    pl.kernel,
    out_type=jax.ShapeDtypeStruct((B * OUT_ROW,), jnp.int32),
    mesh=_mesh,
    compiler_params=pltpu.CompilerParams(needs_layout_passes=False),
    scratch_types=[
        pltpu.VMEM((GRID_PAD,), jnp.int32),
        pltpu.VMEM((2 * ECHUNK,), jnp.int32),
        pltpu.SemaphoreType.DMA,
        pltpu.SemaphoreType.DMA,
        pltpu.SemaphoreType.DMA,
        pltpu.SemaphoreType.DMA,
    ],
)
def _voxelize(enc_hbm, out_hbm, grid_v, buf_v, insem0, insem1, outsem0, outsem1):
    cid = lax.axis_index("c")
    sid = lax.axis_index("s")
    wid = sid * NC + cid

    @pl.when(wid < B)
    def _work():
        b = wid
        zeros16 = jnp.zeros((L,), jnp.int32)
        ones16 = jnp.full((L,), ONE_F32_BITS, jnp.int32)
        insems = (insem0, insem1)
        outsems = (outsem0, outsem1)

        def in_copy(c, bu):
            return pltpu.make_async_copy(
                enc_hbm.at[pl.ds(b * P + c * ECHUNK, ECHUNK)],
                buf_v.at[pl.ds(bu * ECHUNK, ECHUNK)],
                insems[bu])

        def out_copy(c, bu):
            return pltpu.make_async_copy(
                buf_v.at[pl.ds(bu * ECHUNK, ECHUNK)],
                out_hbm.at[pl.ds(b * OUT_ROW + c * XOUT, XOUT)],
                outsems[bu])

        in_copy(0, 0).start()
        in_copy(1, 1).start()

        def zbody(i, carry):
            grid_v[pl.ds(i * L, L)] = zeros16
            return carry
        lax.fori_loop(0, GRID_PAD // L, zbody, 0)

        def scpair(g, carry):
            for bu in (0, 1):
                c = g * 2 + bu
                in_copy(c, bu).wait()

                def vbody(v, carry2, bu=bu):
                    ev = buf_v[pl.ds(bu * ECHUNK + v * L, L)]
                    plsc.store_scatter(grid_v, [ev], ones16, mask=ev >= 0)
                    return carry2
                lax.fori_loop(0, ECHUNK // L, vbody, 0)

                @pl.when(c + 2 < NECH)
                def _(c=c, bu=bu):
                    in_copy(c + 2, bu).start()
            return carry
        lax.fori_loop(0, NECH // 2, scpair, 0)

        def zb(i, carry):
            buf_v[pl.ds(i * L, L)] = zeros16
            return carry
        lax.fori_loop(0, 2 * ECHUNK // L, zb, 0)

        idx0 = lax.iota(jnp.int32, L) * 4

        def fill(c, bu, nvec):
            def fb(v, carry, bu=bu):
                vals = grid_v[pl.ds(c * XCHUNK + v * L, L)]
                plsc.store_scatter(
                    buf_v, [idx0 + (bu * ECHUNK + v * (L * 4))], vals)
                return carry
            lax.fori_loop(0, nvec, fb, 0)

        def xpair(g, carry):
            for bu in (0, 1):
                c = g * 2 + bu

                @pl.when(c >= 2)
                def _(c=c, bu=bu):
                    out_copy(c - 2, bu).wait()

                fill(c, bu, XCHUNK // L)
                out_copy(c, bu).start()
            return carry
        lax.fori_loop(0, NXFULL // 2, xpair, 0)

        out_copy(NXFULL - 2, 0).wait()
        fill(NXFULL, 0, (TAIL_GRID + L - 1) // L)
        tail = pltpu.make_async_copy(
            buf_v.at[pl.ds(0, TAIL_OUT)],
            out_hbm.at[pl.ds(b * OUT_ROW + NXFULL * XOUT, TAIL_OUT)],
            outsems[0])
        tail.start()
        out_copy(NXFULL - 1, 1).wait()
        tail.wait()


def kernel(pointclouds, valid_points):
    enc = _encode(pointclouds, valid_points)
    flat = _voxelize(enc.reshape(B * P))
    return lax.bitcast_convert_type(flat, jnp.float32).reshape(
        B, RES, RES, RES, 4)


# speedup vs baseline: 3.0324x; 2.0897x over previous
import functools

import jax
import jax.numpy as jnp
from jax import lax
from jax.experimental import pallas as pl
from jax.experimental.pallas import tpu as pltpu
from jax.experimental.pallas import tpu_sc as plsc

B = 16
P = 131072
RES = 50
NVOX = RES * RES * RES
GRID_PAD = 125008
OUT_ROW = NVOX * 4

NC, NS, L = 2, 16, 16

BB = 8


def _enc_body(x_ref, y_ref, z_ref, valid_ref, enc_ref):
    def coord(ref):
        t = (ref[...] + 2.0) * 0.25 * 49.0
        ti = jnp.floor(t).astype(jnp.int32)
        return jnp.clip(ti, 0, 49)

    i, j, k = coord(x_ref), coord(y_ref), coord(z_ref)
    lin = (i * 50 + j) * 50 + k
    enc_ref[...] = jnp.where(valid_ref[...], lin, -1)


def _encode(xs, ys, zs, valid):
    spec = pl.BlockSpec((BB, P), lambda b: (b, 0))
    return pl.pallas_call(
        _enc_body,
        grid=(B // BB,),
        in_specs=[spec, spec, spec, spec],
        out_specs=spec,
        out_shape=jax.ShapeDtypeStruct((B, P), jnp.int32),
    )(xs, ys, zs, valid)


ECHUNK = 2048
NECH = P // ECHUNK
XCHUNK = 512
XOUT = XCHUNK * 4
NXFULL = NVOX // XCHUNK
TAIL_GRID = NVOX - NXFULL * XCHUNK
TAIL_OUT = TAIL_GRID * 4

_mesh = plsc.VectorSubcoreMesh(
    core_axis_name="c", subcore_axis_name="s", num_cores=NC, num_subcores=NS)


ONE_F32_BITS = 0x3F800000


@functools.partial(
    pl.kernel,
    out_type=jax.ShapeDtypeStruct((B * OUT_ROW,), jnp.int32),
    mesh=_mesh,
    compiler_params=pltpu.CompilerParams(needs_layout_passes=False),
    scratch_types=[
        pltpu.VMEM((GRID_PAD,), jnp.int32),
        pltpu.VMEM((2 * ECHUNK,), jnp.int32),
        pltpu.SemaphoreType.DMA,
        pltpu.SemaphoreType.DMA,
        pltpu.SemaphoreType.DMA,
        pltpu.SemaphoreType.DMA,
    ],
)
def _voxelize(enc_hbm, out_hbm, grid_v, buf_v, insem0, insem1, outsem0, outsem1):
    cid = lax.axis_index("c")
    sid = lax.axis_index("s")
    wid = sid * NC + cid

    @pl.when(wid < B)
    def _work():
        b = wid
        zeros16 = jnp.zeros((L,), jnp.int32)
        ones16 = jnp.full((L,), ONE_F32_BITS, jnp.int32)
        insems = (insem0, insem1)
        outsems = (outsem0, outsem1)

        def in_copy(c, bu):
            return pltpu.make_async_copy(
                enc_hbm.at[pl.ds(b * P + c * ECHUNK, ECHUNK)],
                buf_v.at[pl.ds(bu * ECHUNK, ECHUNK)],
                insems[bu])

        def out_copy(c, bu):
            return pltpu.make_async_copy(
                buf_v.at[pl.ds(bu * ECHUNK, ECHUNK)],
                out_hbm.at[pl.ds(b * OUT_ROW + c * XOUT, XOUT)],
                outsems[bu])

        in_copy(0, 0).start()
        in_copy(1, 1).start()

        def zbody(i, carry):
            grid_v[pl.ds(i * L, L)] = zeros16
            return carry
        lax.fori_loop(0, GRID_PAD // L, zbody, 0)

        def scpair(g, carry):
            for bu in (0, 1):
                c = g * 2 + bu
                in_copy(c, bu).wait()

                def vbody(v, carry2, bu=bu):
                    ev = buf_v[pl.ds(bu * ECHUNK + v * L, L)]
                    plsc.store_scatter(grid_v, [ev], ones16, mask=ev >= 0)
                    return carry2
                lax.fori_loop(0, ECHUNK // L, vbody, 0)

                @pl.when(c + 2 < NECH)
                def _(c=c, bu=bu):
                    in_copy(c + 2, bu).start()
            return carry
        lax.fori_loop(0, NECH // 2, scpair, 0)

        def zb(i, carry):
            buf_v[pl.ds(i * L, L)] = zeros16
            return carry
        lax.fori_loop(0, 2 * ECHUNK // L, zb, 0)

        idx0 = lax.iota(jnp.int32, L) * 4

        def fill(c, bu, nvec):
            def fb(v, carry, bu=bu):
                vals = grid_v[pl.ds(c * XCHUNK + v * L, L)]
                plsc.store_scatter(
                    buf_v, [idx0 + (bu * ECHUNK + v * (L * 4))], vals)
                return carry
            lax.fori_loop(0, nvec, fb, 0)

        def xpair(g, carry):
            for bu in (0, 1):
                c = g * 2 + bu

                @pl.when(c >= 2)
                def _(c=c, bu=bu):
                    out_copy(c - 2, bu).wait()

                fill(c, bu, XCHUNK // L)
                out_copy(c, bu).start()
            return carry
        lax.fori_loop(0, NXFULL // 2, xpair, 0)

        out_copy(NXFULL - 2, 0).wait()
        fill(NXFULL, 0, (TAIL_GRID + L - 1) // L)
        tail = pltpu.make_async_copy(
            buf_v.at[pl.ds(0, TAIL_OUT)],
            out_hbm.at[pl.ds(b * OUT_ROW + NXFULL * XOUT, TAIL_OUT)],
            outsems[0])
        tail.start()
        out_copy(NXFULL - 1, 1).wait()
        tail.wait()


def kernel(pointclouds, valid_points):
    xs = pointclouds[:, :, 0]
    ys = pointclouds[:, :, 1]
    zs = pointclouds[:, :, 2]
    enc = _encode(xs, ys, zs, valid_points)
    flat = _voxelize(enc.reshape(B * P))
    return lax.bitcast_convert_type(flat, jnp.float32).reshape(
        B, RES, RES, RES, 4)


# speedup vs baseline: 3.2806x; 1.0819x over previous
import functools

import jax
import jax.numpy as jnp
from jax import lax
from jax.experimental import pallas as pl
from jax.experimental.pallas import tpu as pltpu
from jax.experimental.pallas import tpu_sc as plsc

B = 16
P = 131072
RES = 50
NVOX = RES * RES * RES
GRID_PAD = 125008
OUT_ROW = NVOX * 4

NC, NS, L = 2, 16, 16

BB = 8


def _enc_body(x_ref, y_ref, z_ref, valid_ref, enc_ref):
    def coord(ref):
        t = (ref[...] + 2.0) * 0.25 * 49.0
        ti = jnp.floor(t).astype(jnp.int32)
        return jnp.clip(ti, 0, 49)

    i, j, k = coord(x_ref), coord(y_ref), coord(z_ref)
    lin = (i * 50 + j) * 50 + k
    enc_ref[...] = jnp.where(valid_ref[...], lin, -1).reshape(BB * P)


def _encode(xs, ys, zs, valid):
    spec = pl.BlockSpec((BB, P), lambda b: (b, 0))
    return pl.pallas_call(
        _enc_body,
        grid=(B // BB,),
        in_specs=[spec, spec, spec, spec],
        out_specs=pl.BlockSpec((BB * P,), lambda b: (b,)),
        out_shape=jax.ShapeDtypeStruct((B * P,), jnp.int32),
    )(xs, ys, zs, valid)


ECHUNK = 2048
NECH = P // ECHUNK
XCHUNK = 512
XOUT = XCHUNK * 4
NXFULL = NVOX // XCHUNK
TAIL_GRID = NVOX - NXFULL * XCHUNK
TAIL_OUT = TAIL_GRID * 4

_mesh = plsc.VectorSubcoreMesh(
    core_axis_name="c", subcore_axis_name="s", num_cores=NC, num_subcores=NS)


ONE_F32_BITS = 0x3F800000


@functools.partial(
    pl.kernel,
    out_type=jax.ShapeDtypeStruct((B * OUT_ROW,), jnp.int32),
    mesh=_mesh,
    compiler_params=pltpu.CompilerParams(needs_layout_passes=False),
    scratch_types=[
        pltpu.VMEM((GRID_PAD,), jnp.int32),
        pltpu.VMEM((2 * ECHUNK,), jnp.int32),
        pltpu.SemaphoreType.DMA,
        pltpu.SemaphoreType.DMA,
        pltpu.SemaphoreType.DMA,
        pltpu.SemaphoreType.DMA,
    ],
)
def _voxelize(enc_hbm, out_hbm, grid_v, buf_v, insem0, insem1, outsem0, outsem1):
    cid = lax.axis_index("c")
    sid = lax.axis_index("s")
    wid = sid * NC + cid

    @pl.when(wid < B)
    def _work():
        b = wid
        zeros16 = jnp.zeros((L,), jnp.int32)
        ones16 = jnp.full((L,), ONE_F32_BITS, jnp.int32)
        insems = (insem0, insem1)
        outsems = (outsem0, outsem1)

        def in_copy(c, bu):
            return pltpu.make_async_copy(
                enc_hbm.at[pl.ds(b * P + c * ECHUNK, ECHUNK)],
                buf_v.at[pl.ds(bu * ECHUNK, ECHUNK)],
                insems[bu])

        def out_copy(c, bu):
            return pltpu.make_async_copy(
                buf_v.at[pl.ds(bu * ECHUNK, ECHUNK)],
                out_hbm.at[pl.ds(b * OUT_ROW + c * XOUT, XOUT)],
                outsems[bu])

        in_copy(0, 0).start()
        in_copy(1, 1).start()

        def zbody(i, carry):
            grid_v[pl.ds(i * L, L)] = zeros16
            return carry
        lax.fori_loop(0, GRID_PAD // L, zbody, 0, unroll=8)

        def scpair(g, carry):
            for bu in (0, 1):
                c = g * 2 + bu
                in_copy(c, bu).wait()

                def vbody(v, carry2, bu=bu):
                    ev = buf_v[pl.ds(bu * ECHUNK + v * L, L)]
                    plsc.store_scatter(grid_v, [ev], ones16, mask=ev >= 0)
                    return carry2
                lax.fori_loop(0, ECHUNK // L, vbody, 0, unroll=8)

                @pl.when(c + 2 < NECH)
                def _(c=c, bu=bu):
                    in_copy(c + 2, bu).start()
            return carry
        lax.fori_loop(0, NECH // 2, scpair, 0)

        def zb(i, carry):
            buf_v[pl.ds(i * L, L)] = zeros16
            return carry
        lax.fori_loop(0, 2 * ECHUNK // L, zb, 0, unroll=8)

        idx0 = lax.iota(jnp.int32, L) * 4

        def fill(c, bu, nvec):
            def fb(v, carry, bu=bu):
                vals = grid_v[pl.ds(c * XCHUNK + v * L, L)]
                plsc.store_scatter(
                    buf_v, [idx0 + (bu * ECHUNK + v * (L * 4))], vals)
                return carry
            lax.fori_loop(0, nvec, fb, 0, unroll=8)

        def xpair(g, carry):
            for bu in (0, 1):
                c = g * 2 + bu

                @pl.when(c >= 2)
                def _(c=c, bu=bu):
                    out_copy(c - 2, bu).wait()

                fill(c, bu, XCHUNK // L)
                out_copy(c, bu).start()
            return carry
        lax.fori_loop(0, NXFULL // 2, xpair, 0)

        out_copy(NXFULL - 2, 0).wait()
        fill(NXFULL, 0, (TAIL_GRID + L - 1) // L)
        tail = pltpu.make_async_copy(
            buf_v.at[pl.ds(0, TAIL_OUT)],
            out_hbm.at[pl.ds(b * OUT_ROW + NXFULL * XOUT, TAIL_OUT)],
            outsems[0])
        tail.start()
        out_copy(NXFULL - 1, 1).wait()
        tail.wait()


def kernel(pointclouds, valid_points):
    xs = pointclouds[:, :, 0]
    ys = pointclouds[:, :, 1]
    zs = pointclouds[:, :, 2]
    enc = _encode(xs, ys, zs, valid_points)
    flat = _voxelize(enc)
    return lax.bitcast_convert_type(flat, jnp.float32).reshape(
        B, RES, RES, RES, 4)
